# baseline (speedup 1.0000x reference)
import jax
import jax.numpy as jnp
from jax import lax
from jax.experimental import pallas as pl
from jax.experimental.pallas import tpu as pltpu
from jax.experimental.pallas import tpu_sc as plsc

BATCH = 16384
HIST = 50
DIM = 64
LANES = 16
NC = 2
NS = 16
NW = NC * NS
RPW = BATCH // NW
CHUNK = 16
NCHUNK = RPW // CHUNK
IDXC = CHUNK * HIST
SPLITS = [(o, min(128, IDXC - o)) for o in range(0, IDXC, 128)]
HIMASK = -65536


def _rsqrt_vec(ns):
    v = lax.broadcast_in_dim(ns, (LANES,), ())
    bits = plsc.bitcast(v, jnp.int32)
    r = plsc.bitcast(jnp.int32(0x5F3759DF) - (bits >> 1), jnp.float32)
    for _ in range(3):
        r = r * (1.5 - 0.5 * v * r * r)
    return r


def _body(hashes, weights, table, out,
          idx0, idx1, wts0, wts1, gath0, gath1, outb0, outb1,
          gsem0, gsem1, osem0, osem1):
    idx = (idx0, idx1)
    wts = (wts0, wts1)
    gath = (gath0, gath1)
    outb = (outb0, outb1)
    gsems = (gsem0, gsem1)
    osems = (osem0, osem1)
    wid = lax.axis_index("s") * NC + lax.axis_index("c")
    base = wid * RPW

    def start(g, b):
        r0 = base + g * CHUNK
        pltpu.sync_copy(hashes.at[pl.ds(r0 * HIST, IDXC)], idx[b])
        pltpu.sync_copy(weights.at[pl.ds(r0 * DIM, CHUNK * DIM)], wts[b])
        for (o, n) in SPLITS:
            pltpu.async_copy(table.at[idx[b].at[pl.ds(o, n)]],
                             gath[b].at[pl.ds(o, n)], gsems[b])

    def wait_gather(b):
        for (o, n) in SPLITS:
            pltpu.make_async_copy(table.at[idx[b].at[pl.ds(o, n)]],
                                  gath[b].at[pl.ds(o, n)], gsems[b]).wait()

    def flush(g, b):
        r0 = base + g * CHUNK
        pltpu.async_copy(outb[b], out.at[pl.ds(r0 * DIM, CHUNK * DIM)],
                         osems[b])

    def drain_out(g, b):
        r0 = base + g * CHUNK
        pltpu.make_async_copy(outb[b], out.at[pl.ds(r0 * DIM, CHUNK * DIM)],
                              osems[b]).wait()

    ev = lax.iota(jnp.int32, LANES) * 2
    od = ev + 1

    def compute(b):
        gref = gath[b]
        wref = wts[b]
        oref = outb[b]

        def row(i, _):
            rb = i * HIST
            wb = i * DIM
            wv = [wref[pl.ds(wb + k * LANES, LANES)] for k in range(4)]
            ae0 = jnp.zeros((LANES,), jnp.float32)
            ao0 = jnp.zeros((LANES,), jnp.float32)
            ae1 = jnp.zeros((LANES,), jnp.float32)
            ao1 = jnp.zeros((LANES,), jnp.float32)
            for l in range(HIST):
                w = wv[l // LANES][l % LANES]
                w32a = plsc.bitcast(gref[rb + l, pl.ds(0, 2 * LANES)],
                                    jnp.int32)
                w32b = plsc.bitcast(gref[rb + l, pl.ds(2 * LANES, 2 * LANES)],
                                    jnp.int32)
                ae0 = ae0 + w * plsc.bitcast(w32a << 16, jnp.float32)
                ao0 = ao0 + w * plsc.bitcast(w32a & HIMASK, jnp.float32)
                ae1 = ae1 + w * plsc.bitcast(w32b << 16, jnp.float32)
                ao1 = ao1 + w * plsc.bitcast(w32b & HIMASK, jnp.float32)
            ns = jnp.sum(ae0 * ae0 + ao0 * ao0 + ae1 * ae1 + ao1 * ao1)
            r = _rsqrt_vec(ns)
            fb = i * DIM
            plsc.store_scatter(oref, [fb + ev], ae0 * r)
            plsc.store_scatter(oref, [fb + od], ao0 * r)
            plsc.store_scatter(oref, [fb + 2 * LANES + ev], ae1 * r)
            plsc.store_scatter(oref, [fb + 2 * LANES + od], ao1 * r)
            return 0

        lax.fori_loop(0, CHUNK, row, 0)

    start(0, 0)

    def outer(gi, _):
        gbase = gi * 2
        for b in range(2):
            g = gbase + b
            nb = 1 - b

            @pl.when(g + 1 < NCHUNK)
            def _():
                start(g + 1, nb)

            wait_gather(b)

            @pl.when(g >= 2)
            def _():
                drain_out(g - 2, b)

            compute(b)
            flush(g, b)
        return 0

    lax.fori_loop(0, NCHUNK // 2, outer, 0)
    drain_out(NCHUNK - 2, 0)
    drain_out(NCHUNK - 1, 1)


_sc_call = pl.kernel(
    _body,
    out_type=jax.ShapeDtypeStruct((BATCH * DIM,), jnp.float32),
    mesh=plsc.VectorSubcoreMesh(core_axis_name="c", subcore_axis_name="s"),
    compiler_params=pltpu.CompilerParams(needs_layout_passes=False,
                                         use_tc_tiling_on_sc=False),
    scratch_types=[
        pltpu.VMEM((IDXC,), jnp.int32),
        pltpu.VMEM((IDXC,), jnp.int32),
        pltpu.VMEM((CHUNK * DIM,), jnp.float32),
        pltpu.VMEM((CHUNK * DIM,), jnp.float32),
        pltpu.VMEM((IDXC, DIM), jnp.bfloat16),
        pltpu.VMEM((IDXC, DIM), jnp.bfloat16),
        pltpu.VMEM((CHUNK * DIM,), jnp.float32),
        pltpu.VMEM((CHUNK * DIM,), jnp.float32),
        pltpu.SemaphoreType.DMA,
        pltpu.SemaphoreType.DMA,
        pltpu.SemaphoreType.DMA,
        pltpu.SemaphoreType.DMA,
    ],
)


def kernel(feature_hashes, feature_weights, table):
    tb = table.astype(jnp.bfloat16)
    wpad = jnp.pad(feature_weights, ((0, 0), (0, DIM - HIST)))
    flat = _sc_call(feature_hashes.reshape(-1), wpad.reshape(-1), tb)
    return flat.reshape(BATCH, DIM)


# speedup vs baseline: 1.2857x; 1.2857x over previous
import jax
import jax.numpy as jnp
from jax import lax
from jax.experimental import pallas as pl
from jax.experimental.pallas import tpu as pltpu
from jax.experimental.pallas import tpu_sc as plsc

BATCH = 16384
HIST = 50
DIM = 64
LANES = 16
NC = 2
NS = 16
NW = NC * NS
RPW = BATCH // NW
CHUNK = 16
NCHUNK = RPW // CHUNK


def _rsqrt_vec(ns):
    v = lax.broadcast_in_dim(ns, (LANES,), ())
    bits = plsc.bitcast(v, jnp.int32)
    r = plsc.bitcast(jnp.int32(0x5F3759DF) - (bits >> 1), jnp.float32)
    for _ in range(3):
        r = r * (1.5 - 0.5 * v * r * r)
    return r


def _body(hashes, weights, table, out,
          idx0, idx1, wts0, wts1, gath0, gath1, outb0, outb1,
          gsem0, gsem1, osem0, osem1):
    idx = (idx0, idx1)
    wts = (wts0, wts1)
    gath = (gath0, gath1)
    outb = (outb0, outb1)
    gsems = (gsem0, gsem1)
    osems = (osem0, osem1)
    wid = lax.axis_index("s") * NC + lax.axis_index("c")
    base = wid * RPW

    def start(g, b):
        r0 = base + g * CHUNK
        pltpu.sync_copy(hashes.at[pl.ds(r0, CHUNK)], idx[b])
        pltpu.sync_copy(weights.at[pl.ds(r0, CHUNK)], wts[b])
        for i in range(CHUNK):
            pltpu.async_copy(table.at[idx[b].at[i]], gath[b].at[i], gsems[b])

    def wait_gather(b):
        for i in range(CHUNK):
            pltpu.make_async_copy(table.at[idx[b].at[i]], gath[b].at[i],
                                  gsems[b]).wait()

    def flush(g, b):
        r0 = base + g * CHUNK
        pltpu.async_copy(outb[b], out.at[pl.ds(r0, CHUNK)], osems[b])

    def drain_out(g, b):
        r0 = base + g * CHUNK
        pltpu.make_async_copy(outb[b], out.at[pl.ds(r0, CHUNK)],
                              osems[b]).wait()

    iota16 = lax.iota(jnp.int32, LANES)

    def compute(b):
        gref = gath[b]
        wref = wts[b]
        oref = outb[b]

        def row(i, _):
            ivec = lax.broadcast_in_dim(i, (LANES,), ())
            wv = [plsc.load_gather(
                      wref, [ivec, jnp.minimum(k * LANES + iota16, HIST - 1)])
                  for k in range(4)]
            a0 = jnp.zeros((LANES,), jnp.float32)
            a1 = jnp.zeros((LANES,), jnp.float32)
            a2 = jnp.zeros((LANES,), jnp.float32)
            a3 = jnp.zeros((LANES,), jnp.float32)
            for l in range(HIST):
                w = wv[l // LANES][l % LANES]
                a0 = a0 + w * gref[i, l, pl.ds(0, LANES)]
                a1 = a1 + w * gref[i, l, pl.ds(LANES, LANES)]
                a2 = a2 + w * gref[i, l, pl.ds(2 * LANES, LANES)]
                a3 = a3 + w * gref[i, l, pl.ds(3 * LANES, LANES)]
            ns = jnp.sum(a0 * a0 + a1 * a1 + a2 * a2 + a3 * a3)
            r = _rsqrt_vec(ns)
            oref[i, pl.ds(0, LANES)] = a0 * r
            oref[i, pl.ds(LANES, LANES)] = a1 * r
            oref[i, pl.ds(2 * LANES, LANES)] = a2 * r
            oref[i, pl.ds(3 * LANES, LANES)] = a3 * r
            return 0

        lax.fori_loop(0, CHUNK, row, 0)

    start(0, 0)

    def outer(gi, _):
        gbase = gi * 2
        for b in range(2):
            g = gbase + b
            nb = 1 - b

            @pl.when(g + 1 < NCHUNK)
            def _():
                start(g + 1, nb)

            wait_gather(b)

            @pl.when(g >= 2)
            def _():
                drain_out(g - 2, b)

            compute(b)
            flush(g, b)
        return 0

    lax.fori_loop(0, NCHUNK // 2, outer, 0)
    drain_out(NCHUNK - 2, 0)
    drain_out(NCHUNK - 1, 1)


_sc_call = pl.kernel(
    _body,
    out_type=jax.ShapeDtypeStruct((BATCH, DIM), jnp.float32),
    mesh=plsc.VectorSubcoreMesh(core_axis_name="c", subcore_axis_name="s"),
    compiler_params=pltpu.CompilerParams(needs_layout_passes=False,
                                         use_tc_tiling_on_sc=False),
    scratch_types=[
        pltpu.VMEM((CHUNK, HIST), jnp.int32),
        pltpu.VMEM((CHUNK, HIST), jnp.int32),
        pltpu.VMEM((CHUNK, HIST), jnp.float32),
        pltpu.VMEM((CHUNK, HIST), jnp.float32),
        pltpu.VMEM((CHUNK, HIST, DIM), jnp.float32),
        pltpu.VMEM((CHUNK, HIST, DIM), jnp.float32),
        pltpu.VMEM((CHUNK, DIM), jnp.float32),
        pltpu.VMEM((CHUNK, DIM), jnp.float32),
        pltpu.SemaphoreType.DMA,
        pltpu.SemaphoreType.DMA,
        pltpu.SemaphoreType.DMA,
        pltpu.SemaphoreType.DMA,
    ],
)


def kernel(feature_hashes, feature_weights, table):
    return _sc_call(feature_hashes, feature_weights, table)


# speedup vs baseline: 1.3429x; 1.0444x over previous
import jax
import jax.numpy as jnp
from jax import lax
from jax.experimental import pallas as pl
from jax.experimental.pallas import tpu as pltpu
from jax.experimental.pallas import tpu_sc as plsc

BATCH = 16384
HIST = 50
DIM = 64
LANES = 16
NC = 2
NS = 16
NW = NC * NS
RPW = BATCH // NW
CHUNK = 16
NCHUNK = RPW // CHUNK
NIDX = 4
NGB = 2


def _rsqrt_vec(ns):
    v = lax.broadcast_in_dim(ns, (LANES,), ())
    bits = plsc.bitcast(v, jnp.int32)
    r = plsc.bitcast(jnp.int32(0x5F3759DF) - (bits >> 1), jnp.float32)
    for _ in range(3):
        r = r * (1.5 - 0.5 * v * r * r)
    return r


def _body(hashes, weights, table, out,
          idx0, idx1, idx2, idx3, wts0, wts1, wts2, wts3,
          gath0, gath1, outb0, outb1,
          isem0, isem1, isem2, isem3, gsem0, gsem1, osem0, osem1):
    idx = (idx0, idx1, idx2, idx3)
    wts = (wts0, wts1, wts2, wts3)
    gath = (gath0, gath1)
    outb = (outb0, outb1)
    isems = (isem0, isem1, isem2, isem3)
    gsems = (gsem0, gsem1)
    osems = (osem0, osem1)
    wid = lax.axis_index("s") * NC + lax.axis_index("c")
    base = wid * RPW

    def start_idx(g, s):
        r0 = base + g * CHUNK
        pltpu.async_copy(hashes.at[pl.ds(r0, CHUNK)], idx[s], isems[s])
        pltpu.async_copy(weights.at[pl.ds(r0, CHUNK)], wts[s], isems[s])

    def wait_idx(g, s):
        r0 = base + g * CHUNK
        pltpu.make_async_copy(hashes.at[pl.ds(r0, CHUNK)], idx[s],
                              isems[s]).wait()
        pltpu.make_async_copy(weights.at[pl.ds(r0, CHUNK)], wts[s],
                              isems[s]).wait()

    def start_gather(s, b):
        for i in range(CHUNK):
            pltpu.async_copy(table.at[idx[s].at[i]], gath[b].at[i], gsems[b])

    def wait_gather(s, b):
        for i in range(CHUNK):
            pltpu.make_async_copy(table.at[idx[s].at[i]], gath[b].at[i],
                                  gsems[b]).wait()

    def flush(g, b):
        r0 = base + g * CHUNK
        pltpu.async_copy(outb[b], out.at[pl.ds(r0, CHUNK)], osems[b])

    def drain_out(g, b):
        r0 = base + g * CHUNK
        pltpu.make_async_copy(outb[b], out.at[pl.ds(r0, CHUNK)],
                              osems[b]).wait()

    iota16 = lax.iota(jnp.int32, LANES)

    def compute(s, b):
        gref = gath[b]
        wref = wts[s]
        oref = outb[b]

        def row(i, _):
            ivec = lax.broadcast_in_dim(i, (LANES,), ())
            wv = [plsc.load_gather(
                      wref, [ivec, jnp.minimum(k * LANES + iota16, HIST - 1)])
                  for k in range(4)]
            a0 = jnp.zeros((LANES,), jnp.float32)
            a1 = jnp.zeros((LANES,), jnp.float32)
            a2 = jnp.zeros((LANES,), jnp.float32)
            a3 = jnp.zeros((LANES,), jnp.float32)
            for l in range(HIST):
                w = wv[l // LANES][l % LANES]
                a0 = a0 + w * gref[i, l, pl.ds(0, LANES)]
                a1 = a1 + w * gref[i, l, pl.ds(LANES, LANES)]
                a2 = a2 + w * gref[i, l, pl.ds(2 * LANES, LANES)]
                a3 = a3 + w * gref[i, l, pl.ds(3 * LANES, LANES)]
            ns = jnp.sum(a0 * a0 + a1 * a1 + a2 * a2 + a3 * a3)
            r = _rsqrt_vec(ns)
            oref[i, pl.ds(0, LANES)] = a0 * r
            oref[i, pl.ds(LANES, LANES)] = a1 * r
            oref[i, pl.ds(2 * LANES, LANES)] = a2 * r
            oref[i, pl.ds(3 * LANES, LANES)] = a3 * r
            return 0

        lax.fori_loop(0, CHUNK, row, 0)

    for g0 in range(3):
        start_idx(g0, g0)
    wait_idx(0, 0)
    start_gather(0, 0)

    def outer(oi, _):
        gbase = oi * NIDX
        for u in range(NIDX):
            g = gbase + u
            s = u
            b = u % NGB
            sn = (u + 1) % NIDX
            nb = (u + 1) % NGB

            @pl.when(g + 1 < NCHUNK)
            def _():
                wait_idx(g + 1, sn)
                start_gather(sn, nb)

            @pl.when(g + 3 < NCHUNK)
            def _():
                start_idx(g + 3, (u + 3) % NIDX)

            wait_gather(s, b)

            @pl.when(g >= NGB)
            def _():
                drain_out(g - NGB, b)

            compute(s, b)
            flush(g, b)
        return 0

    lax.fori_loop(0, NCHUNK // NIDX, outer, 0)
    drain_out(NCHUNK - 2, 0)
    drain_out(NCHUNK - 1, 1)


_sc_call = pl.kernel(
    _body,
    out_type=jax.ShapeDtypeStruct((BATCH, DIM), jnp.float32),
    mesh=plsc.VectorSubcoreMesh(core_axis_name="c", subcore_axis_name="s"),
    compiler_params=pltpu.CompilerParams(needs_layout_passes=False,
                                         use_tc_tiling_on_sc=False),
    scratch_types=(
        [pltpu.VMEM((CHUNK, HIST), jnp.int32) for _ in range(NIDX)]
        + [pltpu.VMEM((CHUNK, HIST), jnp.float32) for _ in range(NIDX)]
        + [pltpu.VMEM((CHUNK, HIST, DIM), jnp.float32) for _ in range(NGB)]
        + [pltpu.VMEM((CHUNK, DIM), jnp.float32) for _ in range(NGB)]
        + [pltpu.SemaphoreType.DMA for _ in range(NIDX + NGB + NGB)]
    ),
)


def kernel(feature_hashes, feature_weights, table):
    return _sc_call(feature_hashes, feature_weights, table)
